# baseline (speedup 1.0000x reference)
import functools

import jax
import jax.numpy as jnp
from jax import lax
from jax.experimental import pallas as pl
from jax.experimental.pallas import tpu as pltpu
from jax.experimental.pallas import tpu_sc as plsc

LANES = 16
NUM_WORKERS = 32
NB = 4
LN_EPS = 1e-12


def _lane_sum(x):
    idx = lax.iota(jnp.int32, LANES)
    dnums = lax.GatherDimensionNumbers(
        offset_dims=(), collapsed_slice_dims=(0,), start_index_map=(0,))
    for sh in (8, 4, 2, 1):
        perm = lax.gather(x, (idx ^ sh)[:, None], dimension_numbers=dnums,
                          slice_sizes=(1,),
                          mode=lax.GatherScatterMode.PROMISE_IN_BOUNDS)
        x = x + perm
    return x


def _rsqrt16(a):
    bits = lax.bitcast_convert_type(a, jnp.int32)
    seed = jnp.full((LANES,), 0x5F3759DF, jnp.int32) - (bits >> 1)
    y = lax.bitcast_convert_type(seed, jnp.float32)
    for _ in range(3):
        y = y * (1.5 - 0.5 * a * y * y)
    return y


def kernel(input_ids, token_table, pos_table, gamma, beta):
    B, S = input_ids.shape
    V, H = token_table.shape
    SW = S // NUM_WORKERS
    NH = H // LANES
    NG = B // NB
    inv_h = 1.0 / H

    mesh = plsc.VectorSubcoreMesh(core_axis_name="c", subcore_axis_name="s")

    @functools.partial(
        pl.kernel,
        mesh=mesh,
        out_type=jax.ShapeDtypeStruct((B, S, H), jnp.float32),
        scratch_types=[
            pltpu.VMEM((B, SW), jnp.int32),
            pltpu.VMEM((SW, H), jnp.float32),
            pltpu.VMEM((H,), jnp.float32),
            pltpu.VMEM((H,), jnp.float32),
            pltpu.VMEM((NB, SW, H), jnp.float32),
            pltpu.VMEM((NB, SW, H), jnp.float32),
            pltpu.SemaphoreType.DMA,
            pltpu.SemaphoreType.DMA,
            pltpu.SemaphoreType.DMA,
            pltpu.SemaphoreType.DMA,
            pltpu.SemaphoreType.DMA,
        ],
    )
    def run(ids_h, tok_h, pos_h, g_h, bt_h, out_h,
            idx_v, pos_v, g_v, bt_v, rows0, rows1,
            sem, semg0, semg1, semo0, semo1):
        wid = lax.axis_index("s") * 2 + lax.axis_index("c")
        s0 = wid * SW
        idx_descs = [
            pltpu.async_copy(ids_h.at[pl.ds(b * S + s0, SW)], idx_v.at[b], sem)
            for b in range(B)
        ]
        for d in idx_descs:
            d.wait()
        pltpu.sync_copy(pos_h.at[pl.ds(s0, SW)], pos_v)
        pltpu.sync_copy(g_h, g_v)
        pltpu.sync_copy(bt_h, bt_v)

        def gdesc(b, rows_ref, j, semg):
            return pltpu.make_async_copy(
                tok_h.at[idx_v.at[b]], rows_ref.at[j], semg)

        def odesc(b, rows_ref, j, semo):
            return pltpu.make_async_copy(
                rows_ref.at[j], out_h.at[b, pl.ds(s0, SW)], semo)

        def issue_gathers(g, rows_ref, semg):
            for j in range(NB):
                gdesc(g * NB + j, rows_ref, j, semg).start()

        def wait_gathers(g, rows_ref, semg):
            for j in range(NB):
                gdesc(g * NB + j, rows_ref, j, semg).wait()

        def start_drains(g, rows_ref, semo):
            for j in range(NB):
                odesc(g * NB + j, rows_ref, j, semo).start()

        def wait_drains(g, rows_ref, semo):
            for j in range(NB):
                odesc(g * NB + j, rows_ref, j, semo).wait()

        def compute_group(g, rows_ref, handoff):
            def body_r(r, inner):
                acc_s = [jnp.zeros((LANES,), jnp.float32) for _ in range(NB)]
                acc_q = [jnp.zeros((LANES,), jnp.float32) for _ in range(NB)]
                for i in range(NH):
                    sl = pl.ds(i * LANES, LANES)
                    pv = pos_v[r, sl]
                    for j in range(NB):
                        x = rows_ref[j, r, sl] + pv
                        rows_ref[j, r, sl] = x
                        acc_s[j] = acc_s[j] + x
                        acc_q[j] = acc_q[j] + x * x
                mean = [_lane_sum(a) * inv_h for a in acc_s]
                msq = [_lane_sum(a) * inv_h for a in acc_q]
                rstd = [
                    _rsqrt16(jnp.maximum(msq[j] - mean[j] * mean[j], 0.0)
                             + LN_EPS)
                    for j in range(NB)
                ]
                for i in range(NH):
                    sl = pl.ds(i * LANES, LANES)
                    gv = g_v[sl]
                    bv = bt_v[sl]
                    for j in range(NB):
                        x = rows_ref[j, r, sl]
                        rows_ref[j, r, sl] = (x - mean[j]) * rstd[j] * gv + bv

                @pl.when(r == SW // 4 - 1)
                def _():
                    handoff()

                return inner

            lax.fori_loop(0, SW, body_r, 0)

        issue_gathers(0, rows0, semg0)
        issue_gathers(1, rows1, semg1)

        def pair_body(gg, carry):
            for p, mine, semg, semo, other, o_semg, o_semo in (
                (0, rows0, semg0, semo0, rows1, semg1, semo1),
                (1, rows1, semg1, semo1, rows0, semg0, semo0),
            ):
                g = 2 * gg + p
                wait_gathers(g, mine, semg)

                def handoff(g=g, other=other, o_semg=o_semg, o_semo=o_semo):
                    @pl.when(jnp.logical_and(g >= 1, g <= NG - 2))
                    def _():
                        wait_drains(g - 1, other, o_semo)
                        issue_gathers(g + 1, other, o_semg)

                compute_group(g, mine, handoff)
                start_drains(g, mine, semo)
            return carry

        lax.fori_loop(0, NG // 2, pair_body, 0)
        wait_drains(NG - 2, rows0, semo0)
        wait_drains(NG - 1, rows1, semo1)

    return run(input_ids.reshape(-1), token_table, pos_table, gamma, beta)


# speedup vs baseline: 2.3122x; 2.3122x over previous
import functools

import jax
import jax.numpy as jnp
from jax import lax
from jax.experimental import pallas as pl
from jax.experimental.pallas import tpu as pltpu
from jax.experimental.pallas import tpu_sc as plsc

LANES = 16
NUM_WORKERS = 32
NGB = 4
NOB = 2
LN_EPS = 1e-12


def _lane_sum(x):
    idx = lax.iota(jnp.int32, LANES)
    dnums = lax.GatherDimensionNumbers(
        offset_dims=(), collapsed_slice_dims=(0,), start_index_map=(0,))
    for sh in (8, 4, 2, 1):
        perm = lax.gather(x, (idx ^ sh)[:, None], dimension_numbers=dnums,
                          slice_sizes=(1,),
                          mode=lax.GatherScatterMode.PROMISE_IN_BOUNDS)
        x = x + perm
    return x


def _rsqrt16(a):
    bits = lax.bitcast_convert_type(a, jnp.int32)
    seed = jnp.full((LANES,), 0x5F3759DF, jnp.int32) - (bits >> 1)
    y = lax.bitcast_convert_type(seed, jnp.float32)
    for _ in range(3):
        y = y * (1.5 - 0.5 * a * y * y)
    return y


def kernel(input_ids, token_table, pos_table, gamma, beta):
    B, S = input_ids.shape
    V, H = token_table.shape
    SW = S // NUM_WORKERS
    NH = H // LANES
    inv_h = 1.0 / H

    mesh = plsc.VectorSubcoreMesh(core_axis_name="c", subcore_axis_name="s")

    @functools.partial(
        pl.kernel,
        mesh=mesh,
        out_type=jax.ShapeDtypeStruct((B, S, H), jnp.float32),
        scratch_types=[
            pltpu.VMEM((B, SW), jnp.int32),
            pltpu.VMEM((SW, H), jnp.float32),
            pltpu.VMEM((H,), jnp.float32),
            pltpu.VMEM((H,), jnp.float32),
            pltpu.VMEM((NGB, SW, H), jnp.float32),
            pltpu.VMEM((NOB, SW, H), jnp.float32),
            pltpu.SemaphoreType.DMA,
            pltpu.SemaphoreType.DMA,
            pltpu.SemaphoreType.DMA,
            pltpu.SemaphoreType.DMA,
            pltpu.SemaphoreType.DMA,
            pltpu.SemaphoreType.DMA,
            pltpu.SemaphoreType.DMA,
        ],
    )
    def run(ids_h, tok_h, pos_h, g_h, bt_h, out_h,
            idx_v, pos_v, g_v, bt_v, rows_v, outs_v,
            sem, semg0, semg1, semg2, semg3, semo0, semo1):
        semg = [semg0, semg1, semg2, semg3]
        semo = [semo0, semo1]
        wid = lax.axis_index("s") * 2 + lax.axis_index("c")
        s0 = wid * SW
        idx_descs = [
            pltpu.async_copy(ids_h.at[pl.ds(b * S + s0, SW)], idx_v.at[b], sem)
            for b in range(B)
        ]
        for d in idx_descs:
            d.wait()
        pltpu.sync_copy(pos_h.at[pl.ds(s0, SW)], pos_v)
        pltpu.sync_copy(g_h, g_v)
        pltpu.sync_copy(bt_h, bt_v)

        def gdesc(b, k):
            return pltpu.make_async_copy(
                tok_h.at[idx_v.at[b]], rows_v.at[k], semg[k])

        def odesc(b, k):
            return pltpu.make_async_copy(
                outs_v.at[k], out_h.at[b, pl.ds(s0, SW)], semo[k])

        def compute(rows_ref, out_ref):
            def body_r(r, inner):
                acc_s = jnp.zeros((LANES,), jnp.float32)
                acc_q = jnp.zeros((LANES,), jnp.float32)
                for i in range(NH):
                    sl = pl.ds(i * LANES, LANES)
                    x = rows_ref[r, sl] + pos_v[r, sl]
                    rows_ref[r, sl] = x
                    acc_s = acc_s + x
                    acc_q = acc_q + x * x
                mean = _lane_sum(acc_s) * inv_h
                msq = _lane_sum(acc_q) * inv_h
                var = jnp.maximum(msq - mean * mean, 0.0) + LN_EPS
                rstd = _rsqrt16(var)
                for i in range(NH):
                    sl = pl.ds(i * LANES, LANES)
                    x = rows_ref[r, sl]
                    out_ref[r, sl] = (x - mean) * rstd * g_v[sl] + bt_v[sl]
                return inner

            lax.fori_loop(0, SW, body_r, 0)

        for k in range(NGB):
            gdesc(k, k).start()
        n_groups = B // NGB

        def group(g, carry):
            for k in range(NGB):
                b = NGB * g + k
                ko = k % NOB
                gdesc(b, k).wait()
                if k < NOB:
                    @pl.when(g > 0)
                    def _drain():
                        odesc(b, ko).wait()
                else:
                    odesc(b, ko).wait()
                compute(rows_v.at[k], outs_v.at[ko])
                odesc(b, ko).start()

                @pl.when(g < n_groups - 1)
                def _prefetch():
                    gdesc(b + NGB, k).start()

            return carry

        lax.fori_loop(0, n_groups, group, 0)
        odesc(B - 2, 0).wait()
        odesc(B - 1, 1).wait()

    return run(input_ids.reshape(-1), token_table, pos_table, gamma, beta)
